# Initial kernel scaffold; baseline (speedup 1.0000x reference)
import jax
import jax.numpy as jnp
from jax.experimental import pallas as pl


def kernel(x, router_w, router_b, w1, b1, w2, b2):
    raise NotImplementedError("write your pallas kernel here")


# speedup vs baseline: 3.3358x; 3.3358x over previous
import functools

import jax
import jax.numpy as jnp
from jax.experimental import pallas as pl
from jax.experimental.pallas import tpu as pltpu

B = 1
T = 2048
D = 768
E = 8
TOPK = 2
H = 1536

_INTERPRET = False


def _routing_body(x_ref, rw_ref, rb_ref,
                  g_ref, idx_ref, wn_ref, fe_ref, pe_ref, lb_ref, zl_ref, ent_ref):
    h = x_ref[...]
    logits = jnp.dot(h, rw_ref[...], preferred_element_type=jnp.float32)
    logits = logits + rb_ref[...]
    m = jnp.max(logits, axis=-1, keepdims=True)
    ex = jnp.exp(logits - m)
    s = jnp.sum(ex, axis=-1, keepdims=True)
    probs = ex / s
    z = m[:, 0] + jnp.log(s[:, 0])

    iota = jax.lax.broadcasted_iota(jnp.int32, (T, E), 1)
    m1 = jnp.max(probs, axis=-1, keepdims=True)
    idx1 = jnp.min(jnp.where(probs == m1, iota, E), axis=-1, keepdims=True)
    p2 = jnp.where(iota == idx1, -jnp.inf, probs)
    m2 = jnp.max(p2, axis=-1, keepdims=True)
    idx2 = jnp.min(jnp.where(p2 == m2, iota, E), axis=-1, keepdims=True)

    wsum = m1 + m2 + 1e-9
    w1n = m1 / wsum
    w2n = m2 / wsum
    oh1 = (iota == idx1).astype(jnp.float32)
    oh2 = (iota == idx2).astype(jnp.float32)
    g_ref[...] = oh1 * w1n + oh2 * w2n
    idx_ref[...] = jnp.concatenate([idx1, idx2], axis=1)
    wn_ref[...] = jnp.concatenate([w1n, w2n], axis=1)

    fe = jnp.mean(oh1 + oh2, axis=0)[None, :]
    pe = jnp.mean(probs, axis=0)[None, :]
    fe_ref[...] = fe
    pe_ref[...] = pe
    lb_ref[...] = jnp.reshape(-E * jnp.sum(fe * pe), (1, 1))
    zl_ref[...] = jnp.reshape(jnp.mean(z * z), (1, 1))
    ent_ref[...] = jnp.reshape(
        jnp.mean(-jnp.sum(probs * jnp.log(probs + 1e-10), axis=-1)), (1, 1))


def _routing(h, router_w, router_b):
    outs = (
        jax.ShapeDtypeStruct((T, E), jnp.float32),
        jax.ShapeDtypeStruct((T, TOPK), jnp.int32),
        jax.ShapeDtypeStruct((T, TOPK), jnp.float32),
        jax.ShapeDtypeStruct((1, E), jnp.float32),
        jax.ShapeDtypeStruct((1, E), jnp.float32),
        jax.ShapeDtypeStruct((1, 1), jnp.float32),
        jax.ShapeDtypeStruct((1, 1), jnp.float32),
        jax.ShapeDtypeStruct((1, 1), jnp.float32),
    )
    return pl.pallas_call(
        _routing_body,
        out_shape=outs,
        interpret=_INTERPRET,
    )(h, router_w, router_b[None, :])


def _ffn_body(x_ref, g_ref, w1_ref, b1_ref, w2_ref, b2_ref, y_ref):
    e = pl.program_id(1)
    hid = jnp.dot(x_ref[...], w1_ref[0], preferred_element_type=jnp.float32)
    hid = hid + b1_ref[0]
    hid = hid * 0.5 * (1.0 + jax.lax.erf(hid * 0.7071067811865476))
    out = jnp.dot(hid, w2_ref[0], preferred_element_type=jnp.float32)
    out = out + b2_ref[0]
    eiota = jax.lax.broadcasted_iota(jnp.int32, g_ref.shape, 1)
    gcol = jnp.sum(jnp.where(eiota == e, g_ref[...], 0.0), axis=1, keepdims=True)
    out = out * gcol

    @pl.when(e == 0)
    def _init():
        y_ref[...] = out

    @pl.when(e != 0)
    def _acc():
        y_ref[...] += out


def _ffn_dense(h, g, w1, b1, w2, b2, tm=512):
    nt = T // tm
    return pl.pallas_call(
        _ffn_body,
        grid=(nt, E),
        in_specs=[
            pl.BlockSpec((tm, D), lambda t, e: (t, 0)),
            pl.BlockSpec((tm, E), lambda t, e: (t, 0)),
            pl.BlockSpec((1, D, H), lambda t, e: (e, 0, 0)),
            pl.BlockSpec((1, 1, H), lambda t, e: (e, 0, 0)),
            pl.BlockSpec((1, H, D), lambda t, e: (e, 0, 0)),
            pl.BlockSpec((1, 1, D), lambda t, e: (e, 0, 0)),
        ],
        out_specs=pl.BlockSpec((tm, D), lambda t, e: (t, 0)),
        out_shape=jax.ShapeDtypeStruct((T, D), jnp.float32),
        interpret=_INTERPRET,
    )(h, g, w1, b1[:, None, :], w2, b2[:, None, :])


def kernel(x, router_w, router_b, w1, b1, w2, b2):
    h = x.reshape(T, D)
    g, idx, wn, fe, pe, lb, zl, ent = _routing(h, router_w, router_b)
    y = _ffn_dense(h, g, w1, b1, w2, b2)
    return (y.reshape(B, T, D), lb[0, 0], zl[0, 0], ent[0, 0],
            fe[0], pe[0])
